# baseline (speedup 1.0000x reference)
import functools

import jax
import jax.numpy as jnp
from jax import lax
from jax.experimental import pallas as pl
from jax.experimental.pallas import tpu as pltpu
from jax.experimental.pallas import tpu_sc as plsc

N_DENSE = 13
N_SPARSE = 26
VOCAB = 100000
EMBED_DIM = 16
BATCH = 16384

NC = 2
NS = 16
NW = NC * NS
BPW = BATCH // NW
CH = 128
NCHUNK = BPW // CH
NJOB = N_SPARSE * NCHUNK
NFEAT = N_DENSE + N_SPARSE
WBPAD = 48


def _fm_body(xt_hbm, wb_hbm, tbl_hbm, out_hbm,
             xt_v, wb_v, fidx_v, acc_v, gbuf_v, out_v, sem0, sem1):
    cid = lax.axis_index("c")
    sid = lax.axis_index("s")
    wid = sid * NC + cid
    base = wid * BPW

    pltpu.sync_copy(xt_hbm.at[:, pl.ds(base, BPW)], xt_v)
    pltpu.sync_copy(wb_hbm, wb_v)

    zeros16 = jnp.zeros((16,), jnp.float32)

    def zero_body(i, _):
        acc_v[pl.ds(i * 16, 16)] = zeros16
        return 0
    lax.fori_loop(0, BPW, zero_body, 0, unroll=8)

    for f in range(N_SPARSE):
        off = f * VOCAB

        def fill_body(k, _, f=f, off=off):
            v = xt_v[N_DENSE + f, pl.ds(k * 16, 16)].astype(jnp.int32) + off
            fidx_v[f, pl.ds(k * 16, 16)] = v
            return 0
        lax.fori_loop(0, BPW // 16, fill_body, 0, unroll=8)

    sems = (sem0, sem1)

    def gather(j, slot):
        f = j // NCHUNK
        c = j - f * NCHUNK
        return pltpu.make_async_copy(
            tbl_hbm.at[fidx_v.at[f, pl.ds(c * CH, CH)]],
            gbuf_v.at[slot], sems[slot])

    def accumulate(j, slot):
        f = j // NCHUNK
        c = j - f * NCHUNK
        rbase = c * CH

        def row_body(r, _):
            plsc.addupdate(acc_v.at[pl.ds((rbase + r) * 16, 16)],
                           gbuf_v[slot, r, :])
            return 0
        lax.fori_loop(0, CH, row_body, 0, unroll=8)

    gather(0, 0).start()
    gather(1, 1).start()

    def pair_body(p, _):
        j0 = p * 2
        gather(j0, 0).wait()
        accumulate(j0, 0)
        gather(j0 + 2, 0).start()
        gather(j0 + 1, 1).wait()
        accumulate(j0 + 1, 1)
        gather(j0 + 3, 1).start()
        return 0
    lax.fori_loop(0, NJOB // 2 - 1, pair_body, 0)

    gather(NJOB - 2, 0).wait()
    accumulate(NJOB - 2, 0)
    gather(NJOB - 1, 1).wait()
    accumulate(NJOB - 1, 1)

    colbase = jnp.arange(16, dtype=jnp.int32) * 16
    wvecs = [wb_v[pl.ds(16 * t, 16)] for t in range(WBPAD // 16)]

    def bf16_round(v):
        u = plsc.bitcast(v, jnp.int32)
        r = (u + 0x7FFF + ((u >> 16) & 1)) & jnp.int32(-65536)
        return plsc.bitcast(r, jnp.float32)

    def out_body(k, _):
        i0 = k * 16
        a = zeros16 + wvecs[NFEAT // 16][NFEAT % 16]
        for j in range(NFEAT):
            a = a + bf16_round(xt_v[j, pl.ds(i0, 16)]) * wvecs[j // 16][j % 16]
        s1 = zeros16
        s2 = zeros16
        cb = colbase + i0 * 16
        for d in range(EMBED_DIM):
            col = plsc.load_gather(acc_v, [cb + d])
            s1 = s1 + col
            s2 = s2 + col * col
        z = a + 0.5 * (s1 * s1 - s2)
        out_v[pl.ds(i0, 16)] = 1.0 / (1.0 + jnp.exp(-z))
        return 0
    lax.fori_loop(0, BPW // 16, out_body, 0)

    pltpu.sync_copy(out_v, out_hbm.at[pl.ds(base, BPW)])


@jax.jit
def _fm_sc(xt, wb, tbl):
    mesh = plsc.VectorSubcoreMesh(
        core_axis_name="c", subcore_axis_name="s",
        num_cores=NC, num_subcores=NS)
    kfn = pl.kernel(
        _fm_body,
        out_type=jax.ShapeDtypeStruct((BATCH,), jnp.float32),
        mesh=mesh,
        scratch_types=[
            pltpu.VMEM((NFEAT, BPW), jnp.float32),
            pltpu.VMEM((WBPAD,), jnp.float32),
            pltpu.VMEM((N_SPARSE, BPW), jnp.int32),
            pltpu.VMEM((BPW * EMBED_DIM,), jnp.float32),
            pltpu.VMEM((2, CH, EMBED_DIM), jnp.float32),
            pltpu.VMEM((BPW,), jnp.float32),
            pltpu.SemaphoreType.DMA,
            pltpu.SemaphoreType.DMA,
        ],
        compiler_params=pltpu.CompilerParams(
            needs_layout_passes=False, use_tc_tiling_on_sc=False),
    )
    return kfn(xt, wb, tbl)


def kernel(x, tables, W, b):
    xt = x.T
    w_rounded = W[:, 0].astype(jnp.bfloat16).astype(jnp.float32)
    wb = jnp.concatenate([w_rounded, b, jnp.zeros((WBPAD - NFEAT - 1,), jnp.float32)])
    tbl = tables.reshape(N_SPARSE * VOCAB, EMBED_DIM)
    out = _fm_sc(xt, wb, tbl)
    return out[:, None]
